# baseline (speedup 1.0000x reference)
import functools

import jax
import jax.numpy as jnp
from jax import lax
from jax.experimental import pallas as pl
from jax.experimental.pallas import tpu as pltpu
from jax.experimental.pallas import tpu_sc as plsc


def _gather_kernel(B, T, D):
    info = plsc.get_sparse_core_info()
    NC, NS = info.num_cores, info.num_subcores
    NW = NC * NS
    NBUF = 2
    TB = 2
    assert B % NW == 0 and T % (TB * NBUF) == 0
    NB = B // NW
    NCH = T // TB
    LG = 16

    mesh = plsc.VectorSubcoreMesh(core_axis_name="c", subcore_axis_name="s")

    @functools.partial(
        pl.kernel,
        out_type=jax.ShapeDtypeStruct((T, D, B), jnp.float32),
        mesh=mesh,
        scratch_types=[
            [pltpu.VMEM((TB, NB), jnp.int32) for _ in range(NBUF)],
            [pltpu.VMEM((TB, NB, 128), jnp.float32) for _ in range(NBUF)],
            [pltpu.VMEM((TB, D, NB), jnp.float32) for _ in range(NBUF)],
            [pltpu.SemaphoreType.DMA for _ in range(NBUF)],
            [pltpu.SemaphoreType.DMA for _ in range(NBUF)],
        ],
        compiler_params=pltpu.CompilerParams(
            use_tc_tiling_on_sc=True, needs_layout_passes=False
        ),
    )
    def k(tokT_hbm, table_hbm, out_hbm, idx_v, rows_g, rows_t, sem_g, sem_s):
        wid = lax.axis_index("s") * NC + lax.axis_index("c")
        b0 = wid * NB

        def stage_idx(i, b):
            pltpu.sync_copy(
                tokT_hbm.at[pl.ds(i * TB, TB), pl.ds(b0, NB)], idx_v[b]
            )

        def start_gather(b):
            for tb in range(TB):
                pltpu.async_copy(
                    table_hbm.at[idx_v[b].at[tb]], rows_g[b].at[tb], sem_g[b]
                )

        def wait_gather(b):
            for tb in range(TB):
                pltpu.make_async_copy(
                    table_hbm.at[idx_v[b].at[tb]], rows_g[b].at[tb], sem_g[b]
                ).wait()

        def store(i, b, wait):
            for tb in range(TB):
                src = rows_t[b].at[tb]
                dst = out_hbm.at[i * TB + tb, :, pl.ds(b0, NB)]
                if wait:
                    pltpu.make_async_copy(src, dst, sem_s[b]).wait()
                else:
                    pltpu.async_copy(src, dst, sem_s[b])

        tokvecs = [lax.iota(jnp.int32, LG) + g * LG for g in range(NB // LG)]

        def transpose(b):
            for tb in range(TB):
                g_ref, t_ref = rows_g[b].at[tb], rows_t[b].at[tb]

                @pl.loop(0, D, unroll=4)
                def _(d):
                    dvec = jnp.zeros((LG,), jnp.int32) + d
                    for g in range(NB // LG):
                        vals = plsc.load_gather(g_ref, [tokvecs[g], dvec])
                        t_ref[d, pl.ds(g * LG, LG)] = vals

        for b in range(NBUF):
            stage_idx(b, b)
            start_gather(b)

        @pl.loop(0, NCH, step=NBUF)
        def _(g):
            for b in range(NBUF):
                i = g + b
                wait_gather(b)

                @pl.when(i >= NBUF)
                def _():
                    store(i - NBUF, b, wait=True)

                transpose(b)
                store(i, b, wait=False)

                @pl.when(i + NBUF < NCH)
                def _():
                    stage_idx(i + NBUF, b)
                    start_gather(b)

        for b in range(NBUF):
            store(NCH - NBUF + b, b, wait=True)

    return k


def kernel(tokens, embedding_weight):
    B, T = tokens.shape
    V, D = embedding_weight.shape
    k = _gather_kernel(B, T, D)
    tok_t = tokens.T.astype(jnp.int32)
    table_p = jnp.pad(embedding_weight, ((0, 0), (0, 128 - D)))
    out_t = k(tok_t, table_p)
    return jnp.transpose(out_t, (2, 0, 1))


# speedup vs baseline: 1.9105x; 1.9105x over previous
import functools

import jax
import jax.numpy as jnp
from jax import lax
from jax.experimental import pallas as pl
from jax.experimental.pallas import tpu as pltpu
from jax.experimental.pallas import tpu_sc as plsc


def _gather_kernel(N, D, chunk, nbuf):
    info = plsc.get_sparse_core_info()
    NC, NS = info.num_cores, info.num_subcores
    NW = NC * NS
    assert N % (NW * chunk) == 0
    n = N // (NW * chunk)
    assert n >= nbuf
    per_w = N // NW

    mesh = plsc.VectorSubcoreMesh(core_axis_name="c", subcore_axis_name="s")

    @functools.partial(
        pl.kernel,
        out_type=jax.ShapeDtypeStruct((N, D), jnp.float32),
        mesh=mesh,
        scratch_types=[
            [pltpu.VMEM((chunk,), jnp.int32) for _ in range(nbuf)],
            [pltpu.VMEM((chunk, D), jnp.float32) for _ in range(nbuf)],
            [pltpu.SemaphoreType.DMA for _ in range(nbuf)],
            [pltpu.SemaphoreType.DMA for _ in range(nbuf)],
        ],
        compiler_params=pltpu.CompilerParams(use_tc_tiling_on_sc=False),
    )
    def k(idx_hbm, table_hbm, out_hbm, idx_v, rows_v, sem_g, sem_s):
        wid = lax.axis_index("s") * NC + lax.axis_index("c")
        base = wid * per_w

        def stage_idx(c, b):
            pltpu.sync_copy(idx_hbm.at[pl.ds(base + c * chunk, chunk)], idx_v[b])

        def start_gather(b):
            pltpu.async_copy(table_hbm.at[idx_v[b]], rows_v[b], sem_g[b])

        def wait_gather(b):
            pltpu.make_async_copy(table_hbm.at[idx_v[b]], rows_v[b], sem_g[b]).wait()

        def store(c, b, wait):
            src = rows_v[b]
            dst = out_hbm.at[pl.ds(base + c * chunk, chunk)]
            if wait:
                pltpu.make_async_copy(src, dst, sem_s[b]).wait()
            else:
                pltpu.async_copy(src, dst, sem_s[b])

        for b in range(nbuf):
            stage_idx(b, b)
            start_gather(b)

        @pl.loop(0, n, step=nbuf)
        def _(g):
            for b in range(nbuf):
                c = g + b
                wait_gather(b)
                store(c, b, wait=False)

                @pl.when(c + nbuf < n)
                def _():
                    stage_idx(c + nbuf, b)

                store(c, b, wait=True)

                @pl.when(c + nbuf < n)
                def _():
                    start_gather(b)

    return k


def kernel(tokens, embedding_weight):
    B, T = tokens.shape
    V, D = embedding_weight.shape
    k = _gather_kernel(B * T, D, chunk=800, nbuf=2)
    flat_idx = tokens.reshape(B * T).astype(jnp.int32)
    out = k(flat_idx, embedding_weight)
    return out.reshape(B, T, D)
